# Initial kernel scaffold; baseline (speedup 1.0000x reference)
import jax
import jax.numpy as jnp
from jax.experimental import pallas as pl


def kernel(x_room, x_room_virtual, edge_index_rr, edge_index_r_rv, edge_index_rv_r, Wn_0_rr, Wr_0_rr, b_0_rr, Wn_0_r_rv, Wr_0_r_rv, b_0_r_rv, Wn_0_rv_r, Wr_0_rv_r, b_0_rv_r, Wn_1_rr, Wr_1_rr, b_1_rr, Wn_1_r_rv, Wr_1_r_rv, b_1_r_rv, Wn_1_rv_r, Wr_1_rv_r, b_1_rv_r, Wn_2_rr, Wr_2_rr, b_2_rr, Wn_2_r_rv, Wr_2_r_rv, b_2_r_rv, Wn_2_rv_r, Wr_2_rv_r, b_2_rv_r):
    raise NotImplementedError("write your pallas kernel here")


# speedup vs baseline: 2.5443x; 2.5443x over previous
import functools

import jax
import jax.numpy as jnp
from jax import lax
from jax.experimental import pallas as pl
from jax.experimental.pallas import tpu as pltpu
from jax.experimental.pallas import tpu_sc as plsc

N_ROOM = 10000
N_RV = 1000
E_RR = 320000
E_POOL = 10000

NC = 2
NS = 16
NW = NC * NS
CH = 128

N_ACC = 10112
NPO = 1024
CNT_W = 16

G = 8
RR_NTC = 80
PP_NTC = -(-E_POOL // (NW * CH))

_f32 = jnp.float32


def _pad_edges_2d(src, dst, ntc):
    e_pad = NW * ntc * CH
    e = src.shape[0]
    src_p = jnp.concatenate([src, jnp.zeros((e_pad - e,), jnp.int32)])
    dst_p = jnp.concatenate([dst, jnp.full((e_pad - e,), N_ROOM, jnp.int32)])
    return src_p.reshape(NW, ntc, CH), dst_p.reshape(NW, ntc, CH)


def _sc_segsums(width, passes, with_counts):
    np_ = len(passes)
    ntc_max = max(p[3] for p in passes)
    zeros_w = jnp.zeros((N_ACC, width), _f32)
    inputs = []
    for tab, src3, dst3, ntc, orows in passes:
        inputs += [tab, src3, dst3]
    inputs.append(zeros_w)
    if with_counts:
        inputs.append(jnp.ones((CH, CNT_W), _f32))
        inputs.append(jnp.zeros((N_ACC, CNT_W), _f32))

    out_type = [jax.ShapeDtypeStruct((p[4], width), _f32)
                for p in passes for _ in range(NC)]
    if with_counts:
        out_type += [jax.ShapeDtypeStruct((p[4], CNT_W), _f32)
                     for p in passes for _ in range(NC)]

    g_max = min(G, ntc_max)
    scratch = [
        pltpu.VMEM((g_max, CH), jnp.int32),
        pltpu.VMEM((g_max, CH), jnp.int32),
        pltpu.VMEM((CH, width), _f32),
        pltpu.VMEM_SHARED((N_ACC, width), _f32),
    ]
    if with_counts:
        scratch.append(pltpu.VMEM((CH, CNT_W), _f32))
        scratch.append(pltpu.VMEM_SHARED((N_ACC, CNT_W), _f32))
    scratch.append(pltpu.SemaphoreType.DMA)

    def body(*refs):
        it = iter(refs)
        tabs, srcs, dsts = [], [], []
        for _ in range(np_):
            tabs.append(next(it)); srcs.append(next(it)); dsts.append(next(it))
        zw = next(it)
        if with_counts:
            ones_h = next(it); zc = next(it)
        outs = [(next(it), next(it)) for _ in range(np_)]
        couts = [(next(it), next(it)) for _ in range(np_)] if with_counts else []
        sidx = next(it); didx = next(it); rows = next(it); acc = next(it)
        if with_counts:
            onesv = next(it); accc = next(it)
        sem = next(it)

        cid = lax.axis_index("c")
        sid = lax.axis_index("s")
        wid = cid * NS + sid
        zr = N_ACC // NS

        if with_counts:
            pltpu.sync_copy(ones_h, onesv)

        for p in range(np_):
            ntc = passes[p][3]
            orows = passes[p][4]
            pltpu.sync_copy(zw.at[pl.ds(sid * zr, zr)], acc.at[pl.ds(sid * zr, zr)])
            if with_counts:
                pltpu.sync_copy(zc.at[pl.ds(sid * zr, zr)], accc.at[pl.ds(sid * zr, zr)])
            plsc.subcore_barrier()

            tab = tabs[p]
            srcp = srcs[p]
            dstp = dsts[p]
            use_counts = with_counts
            g = min(G, ntc)
            ngroups = ntc // g

            @pl.loop(0, ngroups)
            def _(gi):
                pltpu.sync_copy(srcp.at[wid, pl.ds(gi * g, g)],
                                sidx.at[pl.ds(0, g)])
                pltpu.sync_copy(dstp.at[wid, pl.ds(gi * g, g)],
                                didx.at[pl.ds(0, g)])

                @pl.loop(0, g)
                def _(i):
                    pltpu.async_copy(tab.at[sidx.at[i]], rows, sem).wait()
                    pltpu.sync_copy(rows, acc.at[didx.at[i]], add=True)
                    if use_counts:
                        pltpu.sync_copy(onesv, accc.at[didx.at[i]], add=True)

            plsc.subcore_barrier()
            r = orows // NS
            o0, o1 = outs[p]
            for core, oref in enumerate((o0, o1)):
                @pl.when(cid == core)
                def _(oref=oref):
                    pltpu.sync_copy(acc.at[pl.ds(sid * r, r)],
                                    oref.at[pl.ds(sid * r, r)])
                    if with_counts:
                        pltpu.sync_copy(accc.at[pl.ds(sid * r, r)],
                                        couts[p][core].at[pl.ds(sid * r, r)])
            plsc.subcore_barrier()

    mesh = plsc.VectorSubcoreMesh(core_axis_name="c", subcore_axis_name="s")
    fn = pl.kernel(body, out_type=out_type, mesh=mesh, scratch_types=scratch,
                   compiler_params=pltpu.CompilerParams(use_tc_tiling_on_sc=False))
    res = fn(*inputs)
    if not isinstance(res, (list, tuple)):
        res = [res]
    sums = [(res[2 * p], res[2 * p + 1]) for p in range(np_)]
    if with_counts:
        counts = [(res[2 * np_ + 2 * p], res[2 * np_ + 2 * p + 1])
                  for p in range(np_)]
    else:
        counts = [None] * np_
    return sums, counts


def _mm_multi(x, ws, bs, bm):
    rows, k = x.shape
    grid = rows // bm
    nw = len(ws)

    def bodyf(*refs):
        xr = refs[0]
        wr = refs[1:1 + nw]
        br = refs[1 + nw:1 + 2 * nw]
        outs = refs[1 + 2 * nw:]
        xv = xr[...]
        for i in range(nw):
            outs[i][...] = jnp.dot(xv, wr[i][...],
                                   preferred_element_type=_f32) + br[i][...]

    in_specs = [pl.BlockSpec((bm, k), lambda i: (i, 0))]
    in_specs += [pl.BlockSpec(w.shape, lambda i: (0, 0)) for w in ws]
    in_specs += [pl.BlockSpec((1, w.shape[1]), lambda i: (0, 0)) for w in ws]
    out_specs = [pl.BlockSpec((bm, w.shape[1]), lambda i: (i, 0)) for w in ws]
    out_shape = [jax.ShapeDtypeStruct((rows, w.shape[1]), _f32) for w in ws]
    res = pl.pallas_call(
        bodyf, grid=(grid,), in_specs=in_specs, out_specs=out_specs,
        out_shape=out_shape,
    )(x, *ws, *[b.reshape(1, -1) for b in bs])
    return list(res) if isinstance(res, (list, tuple)) else [res]


def _combine_room(s_a, c_a, s_b, c_b, z, ws, bs, relu, bm):
    rows, w_in = z.shape
    grid = rows // bm
    nw = len(ws)

    def bodyf(*refs):
        (sa0, sa1, ca0, ca1, sb0, sb1, cb0, cb1, zr) = refs[:9]
        wr = refs[9:9 + nw]
        br = refs[9 + nw:9 + 2 * nw]
        outs = refs[9 + 2 * nw:]
        cnt_a = jnp.maximum(ca0[:, :1] + ca1[:, :1], 1.0)
        cnt_b = jnp.maximum(cb0[:, :1] + cb1[:, :1], 1.0)
        x = (sa0[...] + sa1[...]) / cnt_a + (sb0[...] + sb1[...]) / cnt_b + zr[...]
        if relu:
            x = jnp.maximum(x, 0.0)
        if nw == 0:
            outs[0][...] = x
        else:
            for i in range(nw):
                outs[i][...] = jnp.dot(x, wr[i][...],
                                       preferred_element_type=_f32) + br[i][...]

    in_specs = [
        pl.BlockSpec((bm, w_in), lambda i: (i, 0)),
        pl.BlockSpec((bm, w_in), lambda i: (i, 0)),
        pl.BlockSpec((bm, CNT_W), lambda i: (i, 0)),
        pl.BlockSpec((bm, CNT_W), lambda i: (i, 0)),
        pl.BlockSpec((bm, w_in), lambda i: (i, 0)),
        pl.BlockSpec((bm, w_in), lambda i: (i, 0)),
        pl.BlockSpec((bm, CNT_W), lambda i: (i, 0)),
        pl.BlockSpec((bm, CNT_W), lambda i: (i, 0)),
        pl.BlockSpec((bm, w_in), lambda i: (i, 0)),
    ]
    in_specs += [pl.BlockSpec(w.shape, lambda i: (0, 0)) for w in ws]
    in_specs += [pl.BlockSpec((1, w.shape[1]), lambda i: (0, 0)) for w in ws]
    if nw == 0:
        out_specs = [pl.BlockSpec((bm, w_in), lambda i: (i, 0))]
        out_shape = [jax.ShapeDtypeStruct((rows, w_in), _f32)]
    else:
        out_specs = [pl.BlockSpec((bm, w.shape[1]), lambda i: (i, 0)) for w in ws]
        out_shape = [jax.ShapeDtypeStruct((rows, w.shape[1]), _f32) for w in ws]
    res = pl.pallas_call(
        bodyf, grid=(grid,), in_specs=in_specs, out_specs=out_specs,
        out_shape=out_shape,
    )(s_a[0], s_a[1], c_a[0], c_a[1], s_b[0], s_b[1], c_b[0], c_b[1], z,
      *ws, *[b.reshape(1, -1) for b in bs])
    return list(res) if isinstance(res, (list, tuple)) else [res]


def _combine_rv(s, c, z, ws, bs, relu):
    w_in = z.shape[1]
    nw = len(ws)

    def bodyf(*refs):
        s0, s1, c0, c1, zr = refs[:5]
        wr = refs[5:5 + nw]
        br = refs[5 + nw:5 + 2 * nw]
        outs = refs[5 + 2 * nw:]
        ssum = (s0[...] + s1[...])[:N_RV]
        cnt = jnp.maximum((c0[...] + c1[...])[:N_RV, :1], 1.0)
        x = ssum / cnt + zr[...]
        if relu:
            x = jnp.maximum(x, 0.0)
        if nw == 0:
            outs[0][...] = x
        else:
            for i in range(nw):
                outs[i][...] = jnp.dot(x, wr[i][...],
                                       preferred_element_type=_f32) + br[i][...]

    in_specs = [
        pl.BlockSpec((NPO, w_in), lambda i: (0, 0)),
        pl.BlockSpec((NPO, w_in), lambda i: (0, 0)),
        pl.BlockSpec((NPO, CNT_W), lambda i: (0, 0)),
        pl.BlockSpec((NPO, CNT_W), lambda i: (0, 0)),
        pl.BlockSpec((N_RV, w_in), lambda i: (0, 0)),
    ]
    in_specs += [pl.BlockSpec(w.shape, lambda i: (0, 0)) for w in ws]
    in_specs += [pl.BlockSpec((1, w.shape[1]), lambda i: (0, 0)) for w in ws]
    if nw == 0:
        out_specs = [pl.BlockSpec((N_RV, w_in), lambda i: (0, 0))]
        out_shape = [jax.ShapeDtypeStruct((N_RV, w_in), _f32)]
    else:
        out_specs = [pl.BlockSpec((N_RV, w.shape[1]), lambda i: (0, 0)) for w in ws]
        out_shape = [jax.ShapeDtypeStruct((N_RV, w.shape[1]), _f32) for w in ws]
    res = pl.pallas_call(
        bodyf, grid=(1,), in_specs=in_specs, out_specs=out_specs,
        out_shape=out_shape,
    )(s[0], s[1], c[0], c[1], z, *ws, *[b.reshape(1, -1) for b in bs])
    return list(res) if isinstance(res, (list, tuple)) else [res]


def kernel(x_room, x_room_virtual, edge_index_rr, edge_index_r_rv, edge_index_rv_r,
           Wn_0_rr, Wr_0_rr, b_0_rr, Wn_0_r_rv, Wr_0_r_rv, b_0_r_rv, Wn_0_rv_r, Wr_0_rv_r, b_0_rv_r,
           Wn_1_rr, Wr_1_rr, b_1_rr, Wn_1_r_rv, Wr_1_r_rv, b_1_r_rv, Wn_1_rv_r, Wr_1_rv_r, b_1_rv_r,
           Wn_2_rr, Wr_2_rr, b_2_rr, Wn_2_r_rv, Wr_2_r_rv, b_2_r_rv, Wn_2_rv_r, Wr_2_rv_r, b_2_rv_r):
    BM = 1000

    srr2, drr2 = _pad_edges_2d(edge_index_rr[0], edge_index_rr[1], RR_NTC)
    sprv2, dprv2 = _pad_edges_2d(edge_index_r_rv[0], edge_index_r_rv[1], PP_NTC)
    srvr2, drvr2 = _pad_edges_2d(edge_index_rv_r[0], edge_index_rv_r[1], PP_NTC)

    y_rr0, y_prv0, z_room0 = _mm_multi(
        x_room, [Wn_0_rr, Wn_0_r_rv, Wr_0_rr + Wr_0_rv_r],
        [jnp.zeros_like(b_0_rr), jnp.zeros_like(b_0_rr), b_0_rr + b_0_rv_r], BM)
    y_rvr0, z_rv0 = _mm_multi(
        x_room_virtual, [Wn_0_rv_r, Wr_0_r_rv],
        [jnp.zeros_like(b_0_rv_r), b_0_r_rv], N_RV)

    (s_rr0, s_rvr0, s_prv0), (c_rr, c_rvr, c_prv) = _sc_segsums(
        128,
        [(y_rr0, srr2, drr2, RR_NTC, N_ACC),
         (y_rvr0, srvr2, drvr2, PP_NTC, N_ACC),
         (y_prv0, sprv2, dprv2, PP_NTC, NPO)],
        with_counts=True)

    y_rr1, y_prv1, z_room1 = _combine_room(
        s_rr0, c_rr, s_rvr0, c_rvr, z_room0,
        [Wn_1_rr, Wn_1_r_rv, Wr_1_rr + Wr_1_rv_r],
        [jnp.zeros_like(b_1_rr), jnp.zeros_like(b_1_rr), b_1_rr + b_1_rv_r],
        relu=True, bm=BM)
    y_rvr1, z_rv1 = _combine_rv(
        s_prv0, c_prv, z_rv0, [Wn_1_rv_r, Wr_1_r_rv],
        [jnp.zeros_like(b_1_rv_r), b_1_r_rv], relu=True)

    (s_rr1, s_rvr1, s_prv1), _ = _sc_segsums(
        128,
        [(y_rr1, srr2, drr2, RR_NTC, N_ACC),
         (y_rvr1, srvr2, drvr2, PP_NTC, N_ACC),
         (y_prv1, sprv2, dprv2, PP_NTC, NPO)],
        with_counts=False)

    y_rr2, z_room2 = _combine_room(
        s_rr1, c_rr, s_rvr1, c_rvr, z_room1,
        [Wn_2_rr, Wr_2_rr + Wr_2_rv_r],
        [jnp.zeros_like(b_2_rr), b_2_rr + b_2_rv_r], relu=True, bm=BM)
    (y_rvr2,) = _combine_rv(
        s_prv1, c_prv, z_rv1, [Wn_2_rv_r], [jnp.zeros_like(b_2_rv_r)], relu=True)

    (s_rr2, s_rvr2), _ = _sc_segsums(
        32,
        [(y_rr2, srr2, drr2, RR_NTC, N_ACC),
         (y_rvr2, srvr2, drvr2, PP_NTC, N_ACC)],
        with_counts=False)

    (x3,) = _combine_room(s_rr2, c_rr, s_rvr2, c_rvr, z_room2, [], [],
                          relu=False, bm=BM)

    (s_pool,), _ = _sc_segsums(
        32, [(x3, sprv2, dprv2, PP_NTC, NPO)], with_counts=False)

    (out,) = _combine_rv(s_pool, c_prv, jnp.zeros((N_RV, 32), _f32), [], [],
                         relu=False)
    return out
